# Initial kernel scaffold; baseline (speedup 1.0000x reference)
import jax
import jax.numpy as jnp
from jax.experimental import pallas as pl


def kernel(x, weights, connection_indices):
    raise NotImplementedError("write your pallas kernel here")


# speedup vs baseline: 1.7359x; 1.7359x over previous
import functools

import jax
import jax.numpy as jnp
from jax import lax
from jax.experimental import pallas as pl
from jax.experimental.pallas import tpu as pltpu
from jax.experimental.pallas import tpu_sc as plsc

L = 16


@functools.lru_cache(maxsize=None)
def _build(B, IN, OUT, K, NOPS):
    mesh = plsc.VectorSubcoreMesh(core_axis_name="c", subcore_axis_name="s")
    NC, NS = mesh.num_cores, mesh.num_subcores
    NW = NC * NS
    assert B % NW == 0
    rows_per_w = B // NW
    R = 16 if rows_per_w % 16 == 0 else rows_per_w
    n_chunks = rows_per_w // R
    n_groups = OUT // L

    @functools.partial(
        pl.kernel,
        mesh=mesh,
        compiler_params=pltpu.CompilerParams(
            use_tc_tiling_on_sc=False, needs_layout_passes=False
        ),
        out_type=jax.ShapeDtypeStruct((B, OUT), jnp.float32),
        scratch_types=[
            pltpu.VMEM((K, OUT), jnp.int32),
            pltpu.VMEM((NOPS, OUT), jnp.float32),
            pltpu.VMEM((NOPS, OUT), jnp.float32),
            pltpu.VMEM((R, IN), jnp.float32),
            pltpu.VMEM((R, OUT), jnp.float32),
        ],
    )
    def sc_kernel(x_hbm, wt_hbm, idxt_hbm, out_hbm, idx_v, w_v, m_v, xbuf, obuf):
        wid = lax.axis_index("s") * NC + lax.axis_index("c")
        pltpu.sync_copy(idxt_hbm, idx_v)
        pltpu.sync_copy(wt_hbm, w_v)

        one = jnp.full((L,), 1.0, jnp.float32)
        zero = jnp.full((L,), 0.0, jnp.float32)

        def mask_body(g, _):
            s = pl.ds(g * L, L)
            w0, w1, w2, w3 = w_v[0, s], w_v[1, s], w_v[2, s], w_v[3, s]
            m0 = (w0 >= w1) & (w0 >= w2) & (w0 >= w3)
            m1 = (w1 > w0) & (w1 >= w2) & (w1 >= w3)
            m2 = (w2 > w0) & (w2 > w1) & (w2 >= w3)
            m3 = (w3 > w0) & (w3 > w1) & (w3 > w2)
            m_v[0, s] = jnp.where(m0, one, zero)
            m_v[1, s] = jnp.where(m1, one, zero)
            m_v[2, s] = jnp.where(m2, one, zero)
            m_v[3, s] = jnp.where(m3, one, zero)
            return _

        lax.fori_loop(0, n_groups, mask_body, 0, unroll=False)

        for c in range(n_chunks):
            base = wid * rows_per_w + c * R
            pltpu.sync_copy(x_hbm.at[pl.ds(base, R)], xbuf)

            def group_body(g, _):
                s = pl.ds(g * L, L)
                idx = [idx_v[k, s] for k in range(K)]
                m0, m1, m2, m3 = m_v[0, s], m_v[1, s], m_v[2, s], m_v[3, s]
                for r in range(R):
                    rv = jnp.full((L,), r, jnp.int32)
                    gv = [plsc.load_gather(xbuf, [rv, ik]) for ik in idx]
                    mn = gv[0]
                    mx = gv[0]
                    pr = gv[0]
                    q = one - gv[0]
                    for k in range(1, K):
                        mn = jnp.minimum(mn, gv[k])
                        mx = jnp.maximum(mx, gv[k])
                        pr = pr * gv[k]
                        q = q * (one - gv[k])
                    res = m0 * mn + m1 * mx + m2 * pr + m3 * (one - q)
                    obuf[r, s] = res
                return _

            lax.fori_loop(0, n_groups, group_body, 0, unroll=False)
            pltpu.sync_copy(obuf, out_hbm.at[pl.ds(base, R)])

    return sc_kernel


def kernel(x, weights, connection_indices):
    B, IN = x.shape
    OUT, NOPS = weights.shape
    K = connection_indices.shape[1]
    sc_kernel = _build(B, IN, OUT, K, NOPS)
    wt = jnp.transpose(weights)
    idxt = jnp.transpose(connection_indices)
    return sc_kernel(x, wt, idxt)


# speedup vs baseline: 1.8117x; 1.0436x over previous
import functools

import jax
import jax.numpy as jnp
from jax import lax
from jax.experimental import pallas as pl
from jax.experimental.pallas import tpu as pltpu
from jax.experimental.pallas import tpu_sc as plsc

L = 16


def _tree(op, vals):
    vals = list(vals)
    while len(vals) > 1:
        nxt = [op(vals[i], vals[i + 1]) for i in range(0, len(vals) - 1, 2)]
        if len(vals) % 2:
            nxt.append(vals[-1])
        vals = nxt
    return vals[0]


@functools.lru_cache(maxsize=None)
def _build(B, IN, OUT, K, NOPS):
    mesh = plsc.VectorSubcoreMesh(core_axis_name="c", subcore_axis_name="s")
    NC, NS = mesh.num_cores, mesh.num_subcores
    NW = NC * NS
    assert B % NW == 0
    rows_per_w = B // NW
    R = 16 if rows_per_w % 16 == 0 else rows_per_w
    n_chunks = rows_per_w // R
    n_groups = OUT // L

    @functools.partial(
        pl.kernel,
        mesh=mesh,
        compiler_params=pltpu.CompilerParams(
            use_tc_tiling_on_sc=False, needs_layout_passes=False
        ),
        out_type=jax.ShapeDtypeStruct((B, OUT), jnp.float32),
        scratch_types=[
            pltpu.VMEM((K, OUT), jnp.int32),
            pltpu.VMEM((NOPS, OUT), jnp.float32),
            pltpu.VMEM((OUT,), jnp.int32),
            pltpu.VMEM((OUT,), jnp.int32),
            pltpu.VMEM((R, IN), jnp.float32),
            pltpu.VMEM((R, OUT), jnp.float32),
        ],
    )
    def sc_kernel(x_hbm, wt_hbm, idxt_hbm, perm_hbm, out_hbm,
                  idx_v, w_v, id_v, perm_v, xbuf, obuf):
        wid = lax.axis_index("s") * NC + lax.axis_index("c")
        pltpu.sync_copy(idxt_hbm, idx_v)
        pltpu.sync_copy(wt_hbm, w_v)
        pltpu.sync_copy(perm_hbm, perm_v)

        one = jnp.full((L,), 1.0, jnp.float32)
        rows = [jnp.full((L,), r, jnp.int32) for r in range(R)]

        def opid_body(g, _):
            s = pl.ds(g * L, L)
            w0, w1, w2, w3 = w_v[0, s], w_v[1, s], w_v[2, s], w_v[3, s]
            i0 = jnp.full((L,), 0, jnp.int32)
            b1 = w1 > w0
            m01 = jnp.maximum(w0, w1)
            i01 = jnp.where(b1, jnp.full((L,), 1, jnp.int32), i0)
            b2 = w2 > m01
            m012 = jnp.maximum(m01, w2)
            i012 = jnp.where(b2, jnp.full((L,), 2, jnp.int32), i01)
            b3 = w3 > m012
            id_v[s] = jnp.where(b3, jnp.full((L,), 3, jnp.int32), i012)
            return _

        lax.fori_loop(0, n_groups, opid_body, 0, unroll=False)

        for c in range(n_chunks):
            base = wid * rows_per_w + c * R
            pltpu.sync_copy(x_hbm.at[pl.ds(base, R)], xbuf)

            @plsc.parallel_loop(0, n_groups, 1)
            def group_body(g):
                s = pl.ds(g * L, L)
                idx = [idx_v[k, s] for k in range(K)]
                cols = perm_v[s]
                opid = id_v[s]
                sel_min = jnp.min(opid)
                sel_max = jnp.max(opid)
                sel = jnp.where(sel_min == sel_max, sel_min,
                                jnp.int32(NOPS))

                def uniform_arm(redop, post):
                    def arm():
                        for r in range(R):
                            gv = [plsc.load_gather(xbuf.at[r], [ik])
                                  for ik in idx]
                            plsc.store_scatter(
                                obuf, [rows[r], cols], post(_tree(redop, gv)))
                    return arm

                def coein_arm():
                    for r in range(R):
                        gv = [plsc.load_gather(xbuf.at[r], [ik])
                              for ik in idx]
                        q = _tree(lax.mul, [one - v for v in gv])
                        plsc.store_scatter(obuf, [rows[r], cols], one - q)

                def mixed_arm():
                    is_mx = opid == 1
                    is_co = opid == 3
                    is_pc = opid >= 2
                    for r in range(R):
                        gv = [plsc.load_gather(xbuf.at[r], [ik])
                              for ik in idx]
                        mn = _tree(jnp.minimum, gv)
                        mx = _tree(jnp.maximum, gv)
                        pr = _tree(lax.mul, gv)
                        q = _tree(lax.mul, [one - v for v in gv])
                        r01 = jnp.where(is_mx, mx, mn)
                        r23 = jnp.where(is_co, one - q, pr)
                        plsc.store_scatter(
                            obuf, [rows[r], cols], jnp.where(is_pc, r23, r01))

                lax.switch(sel, [
                    uniform_arm(jnp.minimum, lambda v: v),
                    uniform_arm(jnp.maximum, lambda v: v),
                    uniform_arm(lax.mul, lambda v: v),
                    coein_arm,
                    mixed_arm,
                ])
            pltpu.sync_copy(obuf, out_hbm.at[pl.ds(base, R)])

    return sc_kernel


def kernel(x, weights, connection_indices):
    B, IN = x.shape
    OUT, NOPS = weights.shape
    K = connection_indices.shape[1]
    sc_kernel = _build(B, IN, OUT, K, NOPS)
    opid = jnp.argmax(weights, axis=-1)
    perm = jnp.argsort(opid, stable=True).astype(jnp.int32)
    wt = jnp.transpose(weights[perm])
    idxt = jnp.transpose(connection_indices[perm])
    return sc_kernel(x, wt, idxt, perm)
